# Initial kernel scaffold; baseline (speedup 1.0000x reference)
import jax
import jax.numpy as jnp
from jax.experimental import pallas as pl


def kernel(x, edge_index, batch, mf1_Wl, mf1_bl, mf1_Wr, mf2_Wl, mf2_Wr, g1_Wl, g1_Wr, g1_att, g1_b, g2_Wl, g2_Wr, g2_att, g2_b, fc1_W, fc1_b, ln_g, ln_b, fc2_W, fc2_b, fc3_W, fc3_b):
    raise NotImplementedError("write your pallas kernel here")


# speedup vs baseline: 6.1185x; 6.1185x over previous
import functools

import jax
import jax.numpy as jnp
from jax import lax
from jax.experimental import pallas as pl
from jax.experimental.pallas import tpu as pltpu
from jax.experimental.pallas import tpu_sc as plsc

N = 10000
E = 320000
D = 128
H = 64
DEG = 11
G = 64

NC, NS, L = 2, 16, 16
NW = NC * NS
EPT = E // NW
CH = 80
NCHUNK = EPT // CH
STRIPE = N // NS

BN = 400
NB = N // BN

_MESH = plsc.VectorSubcoreMesh(core_axis_name="c", subcore_axis_name="s")


def _zero_rows(ref, nrows, width):
    z = jnp.zeros((16,), jnp.float32)

    def body(i, _):
        for q in range(width // 16):
            ref[i, pl.ds(q * 16, 16)] = z
        return 0

    lax.fori_loop(0, nrows, body, 0)


@functools.partial(
    pl.kernel,
    mesh=_MESH,
    compiler_params=pltpu.CompilerParams(needs_layout_passes=False, use_tc_tiling_on_sc=False),
    out_type=jax.ShapeDtypeStruct((NC, NS, STRIPE, 16), jnp.float32),
    scratch_types=[
        pltpu.VMEM_SHARED((N, 16), jnp.float32),
        pltpu.VMEM((CH, 16), jnp.float32),
        pltpu.VMEM((CH,), jnp.int32),
        pltpu.VMEM((STRIPE, 16), jnp.float32),
    ],
)
def _deg_sc(dst_hbm, out_hbm, acc, ones_v, idx_v, zb):
    core = lax.axis_index("c")
    sid = lax.axis_index("s")
    ebase = (core * NS + sid) * EPT
    onevec = jnp.where(lax.iota(jnp.int32, 16) == 0, 1.0, 0.0)

    def fill1(i, _):
        ones_v[i, :] = onevec
        return 0

    lax.fori_loop(0, CH, fill1, 0)
    _zero_rows(zb, STRIPE, 16)
    pltpu.sync_copy(zb, acc.at[pl.ds(sid * STRIPE, STRIPE), :])
    plsc.subcore_barrier()

    def step(j, _):
        pltpu.sync_copy(dst_hbm.at[pl.ds(ebase + j * CH, CH)], idx_v)
        pltpu.sync_copy(ones_v, acc.at[idx_v], add=True)
        return 0

    lax.fori_loop(0, NCHUNK, step, 0)
    plsc.subcore_barrier()
    pltpu.sync_copy(acc.at[pl.ds(sid * STRIPE, STRIPE), :], zb)
    pltpu.sync_copy(zb, out_hbm.at[core, sid])


@functools.partial(
    pl.kernel,
    mesh=_MESH,
    compiler_params=pltpu.CompilerParams(needs_layout_passes=False, use_tc_tiling_on_sc=False),
    out_type=jax.ShapeDtypeStruct((NC, NS, STRIPE, H), jnp.float32),
    scratch_types=[
        pltpu.VMEM_SHARED((N, H), jnp.float32),
        pltpu.VMEM((N,), jnp.int32),
        pltpu.VMEM((CH,), jnp.int32),
        pltpu.VMEM((CH,), jnp.int32),
        pltpu.VMEM((CH, H), jnp.float32),
        pltpu.VMEM((STRIPE, H), jnp.float32),
        pltpu.SemaphoreType.DMA,
    ],
)
def _mf_sc(table_hbm, src_hbm, dst_hbm, deg_hbm, out_hbm,
           acc, degv, ridx, didx, rows, zb, sem):
    core = lax.axis_index("c")
    sid = lax.axis_index("s")
    ebase = (core * NS + sid) * EPT
    pltpu.sync_copy(deg_hbm, degv)
    _zero_rows(zb, STRIPE, H)
    pltpu.sync_copy(zb, acc.at[pl.ds(sid * STRIPE, STRIPE), :])
    plsc.subcore_barrier()

    def step(j, _):
        pltpu.sync_copy(src_hbm.at[pl.ds(ebase + j * CH, CH)], ridx)
        pltpu.sync_copy(dst_hbm.at[pl.ds(ebase + j * CH, CH)], didx)
        for g in range(CH // 16):
            sl = pl.ds(g * 16, 16)
            dg = plsc.load_gather(degv, [didx[sl]])
            ridx[sl] = ridx[sl] * DEG + dg
        pltpu.async_copy(table_hbm.at[ridx], rows, sem).wait()
        pltpu.sync_copy(rows, acc.at[didx], add=True)
        return 0

    lax.fori_loop(0, NCHUNK, step, 0)
    plsc.subcore_barrier()
    pltpu.sync_copy(acc.at[pl.ds(sid * STRIPE, STRIPE), :], zb)
    pltpu.sync_copy(zb, out_hbm.at[core, sid])


@functools.partial(
    pl.kernel,
    mesh=_MESH,
    compiler_params=pltpu.CompilerParams(needs_layout_passes=False, use_tc_tiling_on_sc=False),
    out_type=jax.ShapeDtypeStruct((NC, NS, STRIPE, 80), jnp.float32),
    scratch_types=[
        pltpu.VMEM_SHARED((N, 80), jnp.float32),
        pltpu.VMEM((CH,), jnp.int32),
        pltpu.VMEM((CH,), jnp.int32),
        pltpu.VMEM((CH, H), jnp.float32),
        pltpu.VMEM((CH, H), jnp.float32),
        pltpu.VMEM((CH, 80), jnp.float32),
        pltpu.VMEM((H,), jnp.float32),
        pltpu.VMEM((STRIPE, 80), jnp.float32),
        pltpu.SemaphoreType.DMA,
        pltpu.SemaphoreType.DMA,
    ],
)
def _gat_sc(xl_hbm, xr_hbm, att_hbm, src_hbm, dst_hbm, out_hbm,
            acc, sidx, didx, xlr, xrr, wr, attv, zb, sem1, sem2):
    core = lax.axis_index("c")
    sid = lax.axis_index("s")
    ebase = (core * NS + sid) * EPT
    pltpu.sync_copy(att_hbm, attv)
    _zero_rows(zb, STRIPE, 80)
    _zero_rows(wr, CH, 80)
    pltpu.sync_copy(zb, acc.at[pl.ds(sid * STRIPE, STRIPE), :])
    plsc.subcore_barrier()

    attvecs = [attv[pl.ds(q * 16, 16)] for q in range(H // 16)]
    att_s = [attvecs[k // 16][k % 16] for k in range(H)]

    def step(j, _):
        pltpu.sync_copy(src_hbm.at[pl.ds(ebase + j * CH, CH)], sidx)
        pltpu.sync_copy(dst_hbm.at[pl.ds(ebase + j * CH, CH)], didx)
        c1 = pltpu.async_copy(xl_hbm.at[sidx], xlr, sem1)
        c2 = pltpu.async_copy(xr_hbm.at[didx], xrr, sem2)
        c1.wait()
        c2.wait()
        for g in range(CH // 16):
            rows16 = lax.iota(jnp.int32, 16) + (g * 16)
            pacc = [jnp.zeros((16,), jnp.float32) for _ in range(4)]
            for k in range(H):
                kv = jnp.full((16,), k, jnp.int32)
                z = (plsc.load_gather(xlr, [rows16, kv])
                     + plsc.load_gather(xrr, [rows16, kv]))
                z = jnp.maximum(z, 0.2 * z)
                pacc[k % 4] = pacc[k % 4] + att_s[k] * z
            a16 = jnp.exp((pacc[0] + pacc[1]) + (pacc[2] + pacc[3]))
            plsc.store_scatter(wr, [rows16, jnp.full((16,), H, jnp.int32)], a16)
            for e in range(16):
                r = g * 16 + e
                a_s = a16[e]
                for q in range(H // 16):
                    wr[r, pl.ds(q * 16, 16)] = a_s * xlr[r, pl.ds(q * 16, 16)]
        pltpu.sync_copy(wr, acc.at[didx], add=True)
        return 0

    lax.fori_loop(0, NCHUNK, step, 0)
    plsc.subcore_barrier()
    pltpu.sync_copy(acc.at[pl.ds(sid * STRIPE, STRIPE), :], zb)
    pltpu.sync_copy(zb, out_hbm.at[core, sid])


def _dot(a, b):
    return lax.dot_general(a, b, (((1,), (0,)), ((), ())),
                           preferred_element_type=jnp.float32,
                           precision=lax.Precision.HIGHEST)


def _elu(v):
    return jnp.where(v > 0, v, jnp.exp(v) - 1.0)


def _mm_body(a_ref, b_ref, o_ref):
    o_ref[...] = _dot(a_ref[...], b_ref[...])


def _mm(a, b):
    n, k = a.shape
    m = b.shape[1]
    return pl.pallas_call(
        _mm_body,
        grid=(n // BN,),
        in_specs=[pl.BlockSpec((BN, k), lambda i: (i, 0)),
                  pl.BlockSpec((k, m), lambda i: (0, 0))],
        out_specs=pl.BlockSpec((BN, m), lambda i: (i, 0)),
        out_shape=jax.ShapeDtypeStruct((n, m), jnp.float32),
    )(a, b)


def _degc_body(a_ref, b_ref, o_ref):
    s = a_ref[:, 0:1] + b_ref[:, 0:1]
    o_ref[...] = jnp.minimum(s, 10.0).astype(jnp.int32)


def _deg_combine(da, db):
    return pl.pallas_call(
        _degc_body,
        grid=(NB,),
        in_specs=[pl.BlockSpec((BN, 16), lambda i: (i, 0)),
                  pl.BlockSpec((BN, 16), lambda i: (i, 0))],
        out_specs=pl.BlockSpec((BN, 1), lambda i: (i, 0)),
        out_shape=jax.ShapeDtypeStruct((N, 1), jnp.int32),
    )(da, db)


def _select_deg(P, deg):
    acc = jnp.zeros((P.shape[0], H), jnp.float32)
    for d in range(DEG):
        m = (deg == d).astype(jnp.float32)
        acc = acc + m * P[:, d * H:(d + 1) * H]
    return acc


def _t2_body(x_ref, h1a_ref, h1b_ref, deg_ref, wr1_ref, bl1_ref, wl2_ref,
             out1_ref, tab2_ref):
    P = _dot(x_ref[...], wr1_ref[...])
    deg = deg_ref[...]
    acc = h1a_ref[...] + h1b_ref[...] + _select_deg(P, deg)
    for d in range(DEG):
        m = (deg == d).astype(jnp.float32)
        acc = acc + m * bl1_ref[d:d + 1, :]
    out1 = _elu(acc)
    out1_ref[...] = out1
    tab2_ref[...] = _dot(out1, wl2_ref[...])


def _t2(x, h1a, h1b, deg, Wr1f, bl1, Wl2f):
    return pl.pallas_call(
        _t2_body,
        grid=(NB,),
        in_specs=[pl.BlockSpec((BN, D), lambda i: (i, 0)),
                  pl.BlockSpec((BN, H), lambda i: (i, 0)),
                  pl.BlockSpec((BN, H), lambda i: (i, 0)),
                  pl.BlockSpec((BN, 1), lambda i: (i, 0)),
                  pl.BlockSpec((D, DEG * H), lambda i: (0, 0)),
                  pl.BlockSpec((DEG, H), lambda i: (0, 0)),
                  pl.BlockSpec((H, DEG * H), lambda i: (0, 0))],
        out_specs=[pl.BlockSpec((BN, H), lambda i: (i, 0)),
                   pl.BlockSpec((BN, DEG * H), lambda i: (i, 0))],
        out_shape=[jax.ShapeDtypeStruct((N, H), jnp.float32),
                   jax.ShapeDtypeStruct((N, DEG * H), jnp.float32)],
    )(x, h1a, h1b, deg, Wr1f, bl1, Wl2f)


def _gat_post(n0, n1, xl, xr, att, b):
    num = n0[:, :H] + n1[:, :H]
    den = n0[:, H:H + 1] + n1[:, H:H + 1]
    z = xl + xr
    z = jnp.maximum(z, 0.2 * z)
    a = jnp.sum(z * att, axis=-1, keepdims=True)
    a = jnp.exp(a)
    num = num + a * xl
    den = den + a
    return _elu(num / (den + 1e-16) + b)


def _t4_body(n0_ref, n1_ref, xl_ref, xr_ref, att_ref, b_ref, wl_ref, wr_ref,
             xl2_ref, xr2_ref):
    xg = _gat_post(n0_ref[...], n1_ref[...], xl_ref[...], xr_ref[...],
                   att_ref[...], b_ref[...])
    xl2_ref[...] = _dot(xg, wl_ref[...])
    xr2_ref[...] = _dot(xg, wr_ref[...])


def _t4(n0, n1, xl, xr, att, b, Wl, Wr):
    return pl.pallas_call(
        _t4_body,
        grid=(NB,),
        in_specs=[pl.BlockSpec((BN, 80), lambda i: (i, 0)),
                  pl.BlockSpec((BN, 80), lambda i: (i, 0)),
                  pl.BlockSpec((BN, H), lambda i: (i, 0)),
                  pl.BlockSpec((BN, H), lambda i: (i, 0)),
                  pl.BlockSpec((1, H), lambda i: (0, 0)),
                  pl.BlockSpec((1, H), lambda i: (0, 0)),
                  pl.BlockSpec((H, H), lambda i: (0, 0)),
                  pl.BlockSpec((H, H), lambda i: (0, 0))],
        out_specs=[pl.BlockSpec((BN, H), lambda i: (i, 0)),
                   pl.BlockSpec((BN, H), lambda i: (i, 0))],
        out_shape=[jax.ShapeDtypeStruct((N, H), jnp.float32),
                   jax.ShapeDtypeStruct((N, H), jnp.float32)],
    )(n0, n1, xl, xr, att, b, Wl, Wr)


def _t5_body(out1_ref, wr2_ref, h2a_ref, h2b_ref, deg_ref, n0_ref, n1_ref,
             xl2_ref, xr2_ref, att2_ref, b2_ref, batch_ref,
             fc1w_ref, fc1b_ref, lng_ref, lnb_ref, fc2w_ref, fc2b_ref,
             fc3w_ref, fc3b_ref, o_ref, pool_ref):
    i = pl.program_id(0)
    Q = _dot(out1_ref[...], wr2_ref[...])
    acc = h2a_ref[...] + h2b_ref[...] + _select_deg(Q, deg_ref[...])
    out2 = _elu(acc)
    xg2 = _gat_post(n0_ref[...], n1_ref[...], xl2_ref[...], xr2_ref[...],
                    att2_ref[...], b2_ref[...])
    s = out2 + xg2
    bt = batch_ref[...].reshape(1, BN)
    gi = lax.broadcasted_iota(jnp.int32, (G, BN), 0)
    oh = (bt == gi).astype(jnp.float32)
    p = lax.dot_general(oh, s, (((1,), (0,)), ((), ())),
                        preferred_element_type=jnp.float32,
                        precision=lax.Precision.HIGHEST)

    @pl.when(i == 0)
    def _():
        pool_ref[...] = p

    @pl.when(i > 0)
    def _():
        pool_ref[...] = pool_ref[...] + p

    @pl.when(i == NB - 1)
    def _():
        pool = pool_ref[...]
        t = _dot(pool, fc1w_ref[...]) + fc1b_ref[...]
        mu = jnp.mean(t, axis=-1, keepdims=True)
        var = jnp.mean((t - mu) ** 2, axis=-1, keepdims=True)
        t = (t - mu) / jnp.sqrt(var + 1e-5) * lng_ref[...] + lnb_ref[...]
        t = jnp.maximum(t, 0.0)
        t = jnp.maximum(_dot(t, fc2w_ref[...]) + fc2b_ref[...], 0.0)
        o_ref[...] = _dot(t, fc3w_ref[...]) + fc3b_ref[...]


def _t5(out1, Wr2f, h2a, h2b, deg, n0, n1, xl2, xr2, att2, b2, batchT,
        fc1_W, fc1_b, ln_g, ln_b, fc2_W, fc2_b, fc3_W, fc3_b):
    full = lambda s: pl.BlockSpec(s, lambda i: tuple(0 for _ in s))
    blk = lambda w: pl.BlockSpec((BN, w), lambda i: (i, 0))
    return pl.pallas_call(
        _t5_body,
        grid=(NB,),
        in_specs=[blk(H), full((H, DEG * H)), blk(H), blk(H), blk(1),
                  blk(80), blk(80), blk(H), blk(H),
                  full((1, H)), full((1, H)),
                  pl.BlockSpec((1, 1, BN), lambda i: (i, 0, 0)),
                  full((H, H)), full((1, H)), full((1, H)), full((1, H)),
                  full((H, 32)), full((1, 32)), full((32, 1)), full((1, 1))],
        out_specs=pl.BlockSpec((G, 1), lambda i: (0, 0)),
        out_shape=jax.ShapeDtypeStruct((G, 1), jnp.float32),
        scratch_shapes=[pltpu.VMEM((G, G), jnp.float32)],
    )(out1, Wr2f, h2a, h2b, deg, n0, n1, xl2, xr2, att2, b2, batchT,
      fc1_W, fc1_b, ln_g, ln_b, fc2_W, fc2_b, fc3_W, fc3_b)


def kernel(x, edge_index, batch, mf1_Wl, mf1_bl, mf1_Wr, mf2_Wl, mf2_Wr,
           g1_Wl, g1_Wr, g1_att, g1_b, g2_Wl, g2_Wr, g2_att, g2_b,
           fc1_W, fc1_b, ln_g, ln_b, fc2_W, fc2_b, fc3_W, fc3_b):
    src = edge_index[0].astype(jnp.int32)
    dst = edge_index[1].astype(jnp.int32)
    Wl1f = mf1_Wl.transpose(1, 0, 2).reshape(D, DEG * H)
    Wr1f = mf1_Wr.transpose(1, 0, 2).reshape(D, DEG * H)
    Wl2f = mf2_Wl.transpose(1, 0, 2).reshape(H, DEG * H)
    Wr2f = mf2_Wr.transpose(1, 0, 2).reshape(H, DEG * H)

    Y = _mm(x, jnp.concatenate([Wl1f, g1_Wl, g1_Wr], axis=1))
    tab1 = Y[:, :DEG * H].reshape(N * DEG, H)
    xl1 = Y[:, DEG * H:DEG * H + H]
    xr1 = Y[:, DEG * H + H:]

    degp = _deg_sc(dst).reshape(NC, N, 16)
    deg = _deg_combine(degp[0], degp[1])
    deg_flat = deg.reshape(N)
    h1p = _mf_sc(tab1, src, dst, deg_flat).reshape(NC, N, H)
    tok1 = h1p[0, 0, 0] * 0.0
    g1p = _gat_sc(xl1, xr1, g1_att + tok1, src, dst).reshape(NC, N, 80)
    tok2 = (g1p[0, 0, 0] * 0.0).astype(jnp.int32)

    out1, tab2w = _t2(x, h1p[0], h1p[1], deg, Wr1f, mf1_bl, Wl2f)
    tab2 = tab2w.reshape(N * DEG, H)

    xl2, xr2 = _t4(g1p[0], g1p[1], xl1, xr1, g1_att.reshape(1, H),
                   g1_b.reshape(1, H), g2_Wl, g2_Wr)

    h2p = _mf_sc(tab2, src, dst, deg_flat + tok2).reshape(NC, N, H)
    tok3 = h2p[0, 0, 0] * 0.0
    g2p = _gat_sc(xl2, xr2, g2_att + tok3, src, dst).reshape(NC, N, 80)

    return _t5(out1, Wr2f, h2p[0], h2p[1], deg, g2p[0], g2p[1], xl2, xr2,
               g2_att.reshape(1, H), g2_b.reshape(1, H),
               batch.astype(jnp.int32).reshape(NB, 1, BN),
               fc1_W, fc1_b.reshape(1, H), ln_g.reshape(1, H),
               ln_b.reshape(1, H), fc2_W, fc2_b.reshape(1, 32),
               fc3_W, fc3_b.reshape(1, 1))


# speedup vs baseline: 7.2428x; 1.1838x over previous
import functools

import jax
import jax.numpy as jnp
from jax import lax
from jax.experimental import pallas as pl
from jax.experimental.pallas import tpu as pltpu
from jax.experimental.pallas import tpu_sc as plsc

N = 10000
E = 320000
D = 128
H = 64
DEG = 11
G = 64

NC, NS, L = 2, 16, 16
NW = NC * NS
EPT = E // NW
CH = 80
GRP = 5
NGRP = EPT // (CH * GRP)
GG = 2
NGG = EPT // (CH * GG)
NTAIL = EPT // CH - NGG * GG
STRIPE = N // NS
ZR = 125

BN = 400
NB = N // BN

_MESH = plsc.VectorSubcoreMesh(core_axis_name="c", subcore_axis_name="s")


def _zero_rows(ref, nrows, width):
    z = jnp.zeros((16,), jnp.float32)

    def body(i, _):
        for q in range(width // 16):
            ref[i, pl.ds(q * 16, 16)] = z
        return 0

    lax.fori_loop(0, nrows, body, 0)


@functools.partial(
    pl.kernel,
    mesh=_MESH,
    compiler_params=pltpu.CompilerParams(needs_layout_passes=False, use_tc_tiling_on_sc=False),
    out_type=jax.ShapeDtypeStruct((NC, NS, STRIPE, 16), jnp.float32),
    scratch_types=[
        pltpu.VMEM_SHARED((N, 16), jnp.float32),
        pltpu.VMEM((CH, 16), jnp.float32),
        pltpu.VMEM((GRP, CH), jnp.int32),
        pltpu.VMEM((STRIPE, 16), jnp.float32),
        pltpu.SemaphoreType.DMA,
        pltpu.SemaphoreType.DMA,
    ],
)
def _deg_sc(dst_hbm, out_hbm, acc, ones_v, idx_v, zb, si, ss):
    core = lax.axis_index("c")
    sid = lax.axis_index("s")
    ebase = (core * NS + sid) * EPT
    onevec = jnp.where(lax.iota(jnp.int32, 16) == 0, 1.0, 0.0)

    def fill1(i, _):
        ones_v[i, :] = onevec
        return 0

    lax.fori_loop(0, CH, fill1, 0)
    _zero_rows(zb, STRIPE, 16)
    pltpu.sync_copy(zb, acc.at[pl.ds(sid * STRIPE, STRIPE), :])
    plsc.subcore_barrier()

    def step(j, _):
        ics = []
        for b in range(GRP):
            c = j * GRP + b
            ics.append(pltpu.async_copy(
                dst_hbm.at[pl.ds(ebase + c * CH, CH)], idx_v.at[b], si))
        for ic in ics:
            ic.wait()
        scs = []
        for b in range(GRP):
            scs.append(pltpu.async_copy(
                ones_v, acc.at[idx_v.at[b]], ss, add=True))
        for s in scs:
            s.wait()
        return 0

    lax.fori_loop(0, NGRP, step, 0)
    plsc.subcore_barrier()
    pltpu.sync_copy(acc.at[pl.ds(sid * STRIPE, STRIPE), :], zb)
    pltpu.sync_copy(zb, out_hbm.at[core, sid])


@functools.partial(
    pl.kernel,
    mesh=_MESH,
    compiler_params=pltpu.CompilerParams(needs_layout_passes=False, use_tc_tiling_on_sc=False),
    out_type=jax.ShapeDtypeStruct((NC, NS, STRIPE, H), jnp.float32),
    scratch_types=[
        pltpu.VMEM_SHARED((N, H), jnp.float32),
        pltpu.VMEM((N,), jnp.int32),
        pltpu.VMEM((GRP, CH), jnp.int32),
        pltpu.VMEM((GRP, CH), jnp.int32),
        pltpu.VMEM((GRP, CH, H), jnp.float32),
        pltpu.VMEM((STRIPE, H), jnp.float32),
        pltpu.SemaphoreType.DMA,
        [pltpu.SemaphoreType.DMA] * GRP,
        pltpu.SemaphoreType.DMA,
    ],
)
def _mf_sc(table_hbm, src_hbm, dst_hbm, deg_hbm, out_hbm,
           acc, degv, ridx, didx, rows, zb, si, sg, ss):
    core = lax.axis_index("c")
    sid = lax.axis_index("s")
    ebase = (core * NS + sid) * EPT
    pltpu.sync_copy(deg_hbm, degv)
    _zero_rows(zb, STRIPE, H)
    pltpu.sync_copy(zb, acc.at[pl.ds(sid * STRIPE, STRIPE), :])
    plsc.subcore_barrier()

    def step(j, _):
        ics = []
        for b in range(GRP):
            c = j * GRP + b
            ics.append(pltpu.async_copy(
                src_hbm.at[pl.ds(ebase + c * CH, CH)], ridx.at[b], si))
            ics.append(pltpu.async_copy(
                dst_hbm.at[pl.ds(ebase + c * CH, CH)], didx.at[b], si))
        for ic in ics:
            ic.wait()
        gcs = []
        for b in range(GRP):
            for g in range(CH // 16):
                sl = pl.ds(g * 16, 16)
                dg = plsc.load_gather(degv, [didx[b, sl]])
                ridx[b, sl] = ridx[b, sl] * DEG + dg
            gcs.append(pltpu.async_copy(
                table_hbm.at[ridx.at[b]], rows.at[b], sg[b]))
        scs = []
        for b in range(GRP):
            gcs[b].wait()
            scs.append(pltpu.async_copy(
                rows.at[b], acc.at[didx.at[b]], ss, add=True))
        for s in scs:
            s.wait()
        return 0

    lax.fori_loop(0, NGRP, step, 0)
    plsc.subcore_barrier()
    pltpu.sync_copy(acc.at[pl.ds(sid * STRIPE, STRIPE), :], zb)
    pltpu.sync_copy(zb, out_hbm.at[core, sid])


@functools.partial(
    pl.kernel,
    mesh=_MESH,
    compiler_params=pltpu.CompilerParams(needs_layout_passes=False, use_tc_tiling_on_sc=False),
    out_type=jax.ShapeDtypeStruct((NC, NS, STRIPE, 80), jnp.float32),
    scratch_types=[
        pltpu.VMEM_SHARED((N, 80), jnp.float32),
        pltpu.VMEM((GG, CH), jnp.int32),
        pltpu.VMEM((GG, CH), jnp.int32),
        pltpu.VMEM((GG, CH, H), jnp.float32),
        pltpu.VMEM((GG, CH, H), jnp.float32),
        pltpu.VMEM((GG, CH, 80), jnp.float32),
        pltpu.VMEM((H,), jnp.float32),
        pltpu.VMEM((ZR, 80), jnp.float32),
        pltpu.SemaphoreType.DMA,
        [pltpu.SemaphoreType.DMA] * GG,
        [pltpu.SemaphoreType.DMA] * GG,
        pltpu.SemaphoreType.DMA,
    ],
)
def _gat_sc(xl_hbm, xr_hbm, att_hbm, src_hbm, dst_hbm, out_hbm,
            acc, sidx, didx, xlr, xrr, wr, attv, zb, si, sg1, sg2, ss):
    core = lax.axis_index("c")
    sid = lax.axis_index("s")
    ebase = (core * NS + sid) * EPT
    pltpu.sync_copy(att_hbm, attv)
    _zero_rows(zb, ZR, 80)
    for b in range(GRP):
        _zero_rows(wr.at[b], CH, 80)
    for p in range(STRIPE // ZR):
        pltpu.sync_copy(zb, acc.at[pl.ds(sid * STRIPE + p * ZR, ZR), :])
    plsc.subcore_barrier()

    attvecs = [attv[pl.ds(q * 16, 16)] for q in range(H // 16)]
    att_s = [attvecs[k // 16][k % 16] for k in range(H)]

    def compute_chunk(b):
        for g in range(CH // 16):
            rows16 = lax.iota(jnp.int32, 16) + (g * 16)
            pacc = [jnp.zeros((16,), jnp.float32) for _ in range(4)]
            for k in range(H):
                kv = jnp.full((16,), k, jnp.int32)
                z = (plsc.load_gather(xlr.at[b], [rows16, kv])
                     + plsc.load_gather(xrr.at[b], [rows16, kv]))
                z = jnp.maximum(z, 0.2 * z)
                pacc[k % 4] = pacc[k % 4] + att_s[k] * z
            a16 = jnp.exp((pacc[0] + pacc[1]) + (pacc[2] + pacc[3]))
            plsc.store_scatter(wr.at[b],
                               [rows16, jnp.full((16,), H, jnp.int32)], a16)
            for e in range(16):
                r = g * 16 + e
                a_s = a16[e]
                for q in range(H // 16):
                    wr[b, r, pl.ds(q * 16, 16)] = (
                        a_s * xlr[b, r, pl.ds(q * 16, 16)])

    def group(cbase, nch):
        ics = []
        for b in range(nch):
            c = cbase + b
            ics.append(pltpu.async_copy(
                src_hbm.at[pl.ds(ebase + c * CH, CH)], sidx.at[b], si))
            ics.append(pltpu.async_copy(
                dst_hbm.at[pl.ds(ebase + c * CH, CH)], didx.at[b], si))
        for ic in ics:
            ic.wait()
        g1s, g2s = [], []
        for b in range(nch):
            g1s.append(pltpu.async_copy(xl_hbm.at[sidx.at[b]], xlr.at[b], sg1[b]))
            g2s.append(pltpu.async_copy(xr_hbm.at[didx.at[b]], xrr.at[b], sg2[b]))
        scs = []
        for b in range(nch):
            g1s[b].wait()
            g2s[b].wait()
            compute_chunk(b)
            scs.append(pltpu.async_copy(
                wr.at[b], acc.at[didx.at[b]], ss, add=True))
        for s in scs:
            s.wait()

    def step(j, _):
        group(j * GG, GG)
        return 0

    lax.fori_loop(0, NGG, step, 0)
    group(NGG * GG, NTAIL)
    plsc.subcore_barrier()
    for p in range(STRIPE // ZR):
        pltpu.sync_copy(acc.at[pl.ds(sid * STRIPE + p * ZR, ZR), :], zb)
        pltpu.sync_copy(zb, out_hbm.at[core, sid, pl.ds(p * ZR, ZR), :])


def _dot(a, b):
    return lax.dot_general(a, b, (((1,), (0,)), ((), ())),
                           preferred_element_type=jnp.float32,
                           precision=lax.Precision.HIGHEST)


def _elu(v):
    return jnp.where(v > 0, v, jnp.exp(v) - 1.0)


def _mm_body(a_ref, b_ref, o_ref):
    o_ref[...] = _dot(a_ref[...], b_ref[...])


def _mm(a, b):
    n, k = a.shape
    m = b.shape[1]
    return pl.pallas_call(
        _mm_body,
        grid=(n // BN,),
        in_specs=[pl.BlockSpec((BN, k), lambda i: (i, 0)),
                  pl.BlockSpec((k, m), lambda i: (0, 0))],
        out_specs=pl.BlockSpec((BN, m), lambda i: (i, 0)),
        out_shape=jax.ShapeDtypeStruct((n, m), jnp.float32),
    )(a, b)


def _degc_body(a_ref, b_ref, o_ref):
    s = a_ref[:, 0:1] + b_ref[:, 0:1]
    o_ref[...] = jnp.minimum(s, 10.0).astype(jnp.int32)


def _deg_combine(da, db):
    return pl.pallas_call(
        _degc_body,
        grid=(NB,),
        in_specs=[pl.BlockSpec((BN, 16), lambda i: (i, 0)),
                  pl.BlockSpec((BN, 16), lambda i: (i, 0))],
        out_specs=pl.BlockSpec((BN, 1), lambda i: (i, 0)),
        out_shape=jax.ShapeDtypeStruct((N, 1), jnp.int32),
    )(da, db)


def _select_deg(P, deg):
    acc = jnp.zeros((P.shape[0], H), jnp.float32)
    for d in range(DEG):
        m = (deg == d).astype(jnp.float32)
        acc = acc + m * P[:, d * H:(d + 1) * H]
    return acc


def _t2_body(x_ref, h1a_ref, h1b_ref, deg_ref, wr1_ref, bl1_ref, wl2_ref,
             out1_ref, tab2_ref):
    P = _dot(x_ref[...], wr1_ref[...])
    deg = deg_ref[...]
    acc = h1a_ref[...] + h1b_ref[...] + _select_deg(P, deg)
    for d in range(DEG):
        m = (deg == d).astype(jnp.float32)
        acc = acc + m * bl1_ref[d:d + 1, :]
    out1 = _elu(acc)
    out1_ref[...] = out1
    tab2_ref[...] = _dot(out1, wl2_ref[...])


def _t2(x, h1a, h1b, deg, Wr1f, bl1, Wl2f):
    return pl.pallas_call(
        _t2_body,
        grid=(NB,),
        in_specs=[pl.BlockSpec((BN, D), lambda i: (i, 0)),
                  pl.BlockSpec((BN, H), lambda i: (i, 0)),
                  pl.BlockSpec((BN, H), lambda i: (i, 0)),
                  pl.BlockSpec((BN, 1), lambda i: (i, 0)),
                  pl.BlockSpec((D, DEG * H), lambda i: (0, 0)),
                  pl.BlockSpec((DEG, H), lambda i: (0, 0)),
                  pl.BlockSpec((H, DEG * H), lambda i: (0, 0))],
        out_specs=[pl.BlockSpec((BN, H), lambda i: (i, 0)),
                   pl.BlockSpec((BN, DEG * H), lambda i: (i, 0))],
        out_shape=[jax.ShapeDtypeStruct((N, H), jnp.float32),
                   jax.ShapeDtypeStruct((N, DEG * H), jnp.float32)],
    )(x, h1a, h1b, deg, Wr1f, bl1, Wl2f)


def _gat_post(n0, n1, xl, xr, att, b):
    num = n0[:, :H] + n1[:, :H]
    den = n0[:, H:H + 1] + n1[:, H:H + 1]
    z = xl + xr
    z = jnp.maximum(z, 0.2 * z)
    a = jnp.sum(z * att, axis=-1, keepdims=True)
    a = jnp.exp(a)
    num = num + a * xl
    den = den + a
    return _elu(num / (den + 1e-16) + b)


def _t4_body(n0_ref, n1_ref, xl_ref, xr_ref, att_ref, b_ref, wl_ref, wr_ref,
             xl2_ref, xr2_ref):
    xg = _gat_post(n0_ref[...], n1_ref[...], xl_ref[...], xr_ref[...],
                   att_ref[...], b_ref[...])
    xl2_ref[...] = _dot(xg, wl_ref[...])
    xr2_ref[...] = _dot(xg, wr_ref[...])


def _t4(n0, n1, xl, xr, att, b, Wl, Wr):
    return pl.pallas_call(
        _t4_body,
        grid=(NB,),
        in_specs=[pl.BlockSpec((BN, 80), lambda i: (i, 0)),
                  pl.BlockSpec((BN, 80), lambda i: (i, 0)),
                  pl.BlockSpec((BN, H), lambda i: (i, 0)),
                  pl.BlockSpec((BN, H), lambda i: (i, 0)),
                  pl.BlockSpec((1, H), lambda i: (0, 0)),
                  pl.BlockSpec((1, H), lambda i: (0, 0)),
                  pl.BlockSpec((H, H), lambda i: (0, 0)),
                  pl.BlockSpec((H, H), lambda i: (0, 0))],
        out_specs=[pl.BlockSpec((BN, H), lambda i: (i, 0)),
                   pl.BlockSpec((BN, H), lambda i: (i, 0))],
        out_shape=[jax.ShapeDtypeStruct((N, H), jnp.float32),
                   jax.ShapeDtypeStruct((N, H), jnp.float32)],
    )(n0, n1, xl, xr, att, b, Wl, Wr)


def _t5_body(out1_ref, wr2_ref, h2a_ref, h2b_ref, deg_ref, n0_ref, n1_ref,
             xl2_ref, xr2_ref, att2_ref, b2_ref, batch_ref,
             fc1w_ref, fc1b_ref, lng_ref, lnb_ref, fc2w_ref, fc2b_ref,
             fc3w_ref, fc3b_ref, o_ref, pool_ref):
    i = pl.program_id(0)
    Q = _dot(out1_ref[...], wr2_ref[...])
    acc = h2a_ref[...] + h2b_ref[...] + _select_deg(Q, deg_ref[...])
    out2 = _elu(acc)
    xg2 = _gat_post(n0_ref[...], n1_ref[...], xl2_ref[...], xr2_ref[...],
                    att2_ref[...], b2_ref[...])
    s = out2 + xg2
    bt = batch_ref[...].reshape(1, BN)
    gi = lax.broadcasted_iota(jnp.int32, (G, BN), 0)
    oh = (bt == gi).astype(jnp.float32)
    p = lax.dot_general(oh, s, (((1,), (0,)), ((), ())),
                        preferred_element_type=jnp.float32,
                        precision=lax.Precision.HIGHEST)

    @pl.when(i == 0)
    def _():
        pool_ref[...] = p

    @pl.when(i > 0)
    def _():
        pool_ref[...] = pool_ref[...] + p

    @pl.when(i == NB - 1)
    def _():
        pool = pool_ref[...]
        t = _dot(pool, fc1w_ref[...]) + fc1b_ref[...]
        mu = jnp.mean(t, axis=-1, keepdims=True)
        var = jnp.mean((t - mu) ** 2, axis=-1, keepdims=True)
        t = (t - mu) / jnp.sqrt(var + 1e-5) * lng_ref[...] + lnb_ref[...]
        t = jnp.maximum(t, 0.0)
        t = jnp.maximum(_dot(t, fc2w_ref[...]) + fc2b_ref[...], 0.0)
        o_ref[...] = _dot(t, fc3w_ref[...]) + fc3b_ref[...]


def _t5(out1, Wr2f, h2a, h2b, deg, n0, n1, xl2, xr2, att2, b2, batchT,
        fc1_W, fc1_b, ln_g, ln_b, fc2_W, fc2_b, fc3_W, fc3_b):
    full = lambda s: pl.BlockSpec(s, lambda i: tuple(0 for _ in s))
    blk = lambda w: pl.BlockSpec((BN, w), lambda i: (i, 0))
    return pl.pallas_call(
        _t5_body,
        grid=(NB,),
        in_specs=[blk(H), full((H, DEG * H)), blk(H), blk(H), blk(1),
                  blk(80), blk(80), blk(H), blk(H),
                  full((1, H)), full((1, H)),
                  pl.BlockSpec((1, 1, BN), lambda i: (i, 0, 0)),
                  full((H, H)), full((1, H)), full((1, H)), full((1, H)),
                  full((H, 32)), full((1, 32)), full((32, 1)), full((1, 1))],
        out_specs=pl.BlockSpec((G, 1), lambda i: (0, 0)),
        out_shape=jax.ShapeDtypeStruct((G, 1), jnp.float32),
        scratch_shapes=[pltpu.VMEM((G, G), jnp.float32)],
    )(out1, Wr2f, h2a, h2b, deg, n0, n1, xl2, xr2, att2, b2, batchT,
      fc1_W, fc1_b, ln_g, ln_b, fc2_W, fc2_b, fc3_W, fc3_b)


def kernel(x, edge_index, batch, mf1_Wl, mf1_bl, mf1_Wr, mf2_Wl, mf2_Wr,
           g1_Wl, g1_Wr, g1_att, g1_b, g2_Wl, g2_Wr, g2_att, g2_b,
           fc1_W, fc1_b, ln_g, ln_b, fc2_W, fc2_b, fc3_W, fc3_b):
    src = edge_index[0].astype(jnp.int32)
    dst = edge_index[1].astype(jnp.int32)
    Wl1f = mf1_Wl.transpose(1, 0, 2).reshape(D, DEG * H)
    Wr1f = mf1_Wr.transpose(1, 0, 2).reshape(D, DEG * H)
    Wl2f = mf2_Wl.transpose(1, 0, 2).reshape(H, DEG * H)
    Wr2f = mf2_Wr.transpose(1, 0, 2).reshape(H, DEG * H)

    Y = _mm(x, jnp.concatenate([Wl1f, g1_Wl, g1_Wr], axis=1))
    tab1 = Y[:, :DEG * H].reshape(N * DEG, H)
    xl1 = Y[:, DEG * H:DEG * H + H]
    xr1 = Y[:, DEG * H + H:]

    degp = _deg_sc(dst).reshape(NC, N, 16)
    deg = _deg_combine(degp[0], degp[1])
    deg_flat = deg.reshape(N)
    h1p = _mf_sc(tab1, src, dst, deg_flat).reshape(NC, N, H)
    tok1 = h1p[0, 0, 0] * 0.0
    g1p = _gat_sc(xl1, xr1, g1_att + tok1, src, dst).reshape(NC, N, 80)
    tok2 = (g1p[0, 0, 0] * 0.0).astype(jnp.int32)

    out1, tab2w = _t2(x, h1p[0], h1p[1], deg, Wr1f, mf1_bl, Wl2f)
    tab2 = tab2w.reshape(N * DEG, H)

    xl2, xr2 = _t4(g1p[0], g1p[1], xl1, xr1, g1_att.reshape(1, H),
                   g1_b.reshape(1, H), g2_Wl, g2_Wr)

    h2p = _mf_sc(tab2, src, dst, deg_flat + tok2).reshape(NC, N, H)
    tok3 = h2p[0, 0, 0] * 0.0
    g2p = _gat_sc(xl2, xr2, g2_att + tok3, src, dst).reshape(NC, N, 80)

    return _t5(out1, Wr2f, h2p[0], h2p[1], deg, g2p[0], g2p[1], xl2, xr2,
               g2_att.reshape(1, H), g2_b.reshape(1, H),
               batch.astype(jnp.int32).reshape(NB, 1, BN),
               fc1_W, fc1_b.reshape(1, H), ln_g.reshape(1, H),
               ln_b.reshape(1, H), fc2_W, fc2_b.reshape(1, 32),
               fc3_W, fc3_b.reshape(1, 1))
